# baseline (speedup 1.0000x reference)
import jax
import jax.numpy as jnp
from jax import lax
from jax.experimental import pallas as pl
from jax.experimental.pallas import tpu as pltpu
from jax.experimental.pallas import tpu_sc as plsc

NUM_MODELS = 8
NUM_SPECIES = 4
DIMS = [384, 160, 128, 96, 1]
N = 20000
B = 1024
G = 23
TOTAL = G * B

NC, NS = 2, 16
NW = NC * NS
APW = N // NW
CH = 125
NCH = APW // CH


def _dot(a, b, prec):
    return lax.dot_general(a, b, (((1,), (0,)), ((), ())),
                           precision=prec,
                           preferred_element_type=jnp.float32)


_PREC = lax.Precision.DEFAULT


def _celu(x):
    return jnp.where(x > 0, x, jnp.exp(x) - 1.0)


def _sc_scatter_body(lin_hbm, pos_hbm, aev_hbm, out_hbm, lin_v, pos_v,
                     buf0, buf1, isem0, isem1, osem0, osem1):
    wid = lax.axis_index("s") * NC + lax.axis_index("c")
    pltpu.sync_copy(lin_hbm.at[wid], lin_v)
    pltpu.sync_copy(pos_hbm.at[wid], pos_v)
    bufs = (buf0, buf1)
    isems = (isem0, isem1)
    osems = (osem0, osem1)
    icp = [None, None]
    ocp = [None, None]
    icp[0] = pltpu.async_copy(aev_hbm.at[lin_v.at[0]], bufs[0], isems[0])
    for c in range(NCH):
        b = c & 1
        icp[b].wait()
        if c + 1 < NCH:
            if c >= 1:
                ocp[1 - b].wait()
            icp[1 - b] = pltpu.async_copy(
                aev_hbm.at[lin_v.at[c + 1]], bufs[1 - b], isems[1 - b])
        ocp[b] = pltpu.async_copy(bufs[b], out_hbm.at[pos_v.at[c]], osems[b])
    ocp[(NCH - 1) & 1].wait()
    if NCH > 1:
        ocp[NCH & 1].wait()


def _sc_scatter(lin3, pos3, aev_flat):
    fn = pl.kernel(
        _sc_scatter_body,
        mesh=plsc.VectorSubcoreMesh(core_axis_name="c", subcore_axis_name="s"),
        out_type=jax.ShapeDtypeStruct((TOTAL, DIMS[0]), jnp.float32),
        scratch_types=[
            pltpu.VMEM((NCH, CH), jnp.int32),
            pltpu.VMEM((NCH, CH), jnp.int32),
            pltpu.VMEM((CH, DIMS[0]), jnp.float32),
            pltpu.VMEM((CH, DIMS[0]), jnp.float32),
            pltpu.SemaphoreType.DMA,
            pltpu.SemaphoreType.DMA,
            pltpu.SemaphoreType.DMA,
            pltpu.SemaphoreType.DMA,
        ],
    )
    return fn(lin3, pos3, aev_flat)


def _mlp_body(bs_ref, bv_ref, x_ref, w0_ref, b0_ref, w1_ref, b1_ref,
              w2_ref, b2_ref, w3_ref, b3_ref, out_ref):
    i = pl.program_id(0)
    nv = bv_ref[i]

    @pl.when(i == 0)
    def _():
        out_ref[...] = jnp.zeros((1, 1), jnp.float32)

    @pl.when(nv > 0)
    def _():
        x = x_ref[...]
        rowmask = lax.broadcasted_iota(jnp.int32, (B, 1), 0) < nv
        h0 = _dot(x, w0_ref[0], _PREC) + b0_ref[0]
        y = jnp.zeros((B, 1), jnp.float32)
        bias_tot = jnp.float32(0.0)
        for m in range(NUM_MODELS):
            h = _celu(h0[:, m * DIMS[1]:(m + 1) * DIMS[1]])
            h = _dot(h, w1_ref[0, m], _PREC) + b1_ref[0, m]
            h = _celu(h)
            h = _dot(h, w2_ref[0, m], _PREC) + b2_ref[0, m]
            h = _celu(h)
            y = y + _dot(h, w3_ref[0, m], _PREC)
            bias_tot = bias_tot + b3_ref[0, m, 0, 0]
        ysum = jnp.sum(jnp.where(rowmask, y, 0.0), axis=(0, 1), keepdims=True)
        acc = ysum + nv.astype(jnp.float32) * bias_tot
        out_ref[...] += acc * (1.0 / NUM_MODELS)


def _routing(species):
    sf = species.reshape(-1).astype(jnp.int32)
    oh = (sf[:, None] == jnp.arange(NUM_SPECIES, dtype=jnp.int32)[None, :]).astype(jnp.int32)
    csum = jnp.cumsum(oh, axis=0)
    counts = csum[-1]
    rank = jnp.sum(oh * csum, axis=1) - 1
    nb = (counts + B - 1) // B
    blk_start = jnp.concatenate([jnp.zeros(1, jnp.int32), jnp.cumsum(nb)])[:NUM_SPECIES]
    pos = jnp.sum(oh * (blk_start * B)[None, :], axis=1) + rank
    g = jnp.arange(G, dtype=jnp.int32)
    cnb = jnp.cumsum(nb)
    bs = jnp.clip(jnp.sum((g[:, None] >= cnb[None, :]).astype(jnp.int32), axis=1),
                  0, NUM_SPECIES - 1).astype(jnp.int32)
    ohg = (bs[:, None] == jnp.arange(NUM_SPECIES, dtype=jnp.int32)[None, :]).astype(jnp.int32)
    counts_g = jnp.sum(ohg * counts[None, :], axis=1)
    start_g = jnp.sum(ohg * blk_start[None, :], axis=1)
    bv = jnp.clip(counts_g - (g - start_g) * B, 0, B).astype(jnp.int32)
    return pos, bs, bv


def kernel(species, aev,
           W0_0, b0_0, W0_1, b0_1, W0_2, b0_2, W0_3, b0_3,
           W1_0, b1_0, W1_1, b1_1, W1_2, b1_2, W1_3, b1_3,
           W2_0, b2_0, W2_1, b2_1, W2_2, b2_2, W2_3, b2_3,
           W3_0, b3_0, W3_1, b3_1, W3_2, b3_2, W3_3, b3_3):
    kw = dict(locals())
    aev_flat = aev.reshape(-1, DIMS[0])
    pos, bs, bv = _routing(species)

    lin = jnp.arange(N, dtype=jnp.int32).reshape(NW, NCH, CH)
    x_sorted = _sc_scatter(lin, pos.reshape(NW, NCH, CH), aev_flat)

    Ws = [jnp.stack([kw["W%d_%d" % (s, l)] for s in range(NUM_SPECIES)])
          for l in range(4)]
    Bs = [jnp.stack([kw["b%d_%d" % (s, l)] for s in range(NUM_SPECIES)])
          for l in range(4)]
    Ws[0] = Ws[0].transpose(0, 2, 1, 3).reshape(NUM_SPECIES, DIMS[0],
                                                NUM_MODELS * DIMS[1])
    Bs[0] = Bs[0].reshape(NUM_SPECIES, 1, NUM_MODELS * DIMS[1])

    def wspec(l):
        return pl.BlockSpec((1, NUM_MODELS, DIMS[l], DIMS[l + 1]),
                            lambda i, bsr, bvr: (bsr[i], 0, 0, 0))

    def bspec(l):
        return pl.BlockSpec((1, NUM_MODELS, 1, DIMS[l + 1]),
                            lambda i, bsr, bvr: (bsr[i], 0, 0, 0))

    in_specs = [pl.BlockSpec((B, DIMS[0]), lambda i, bsr, bvr: (i, 0)),
                pl.BlockSpec((1, DIMS[0], NUM_MODELS * DIMS[1]),
                             lambda i, bsr, bvr: (bsr[i], 0, 0)),
                pl.BlockSpec((1, 1, NUM_MODELS * DIMS[1]),
                             lambda i, bsr, bvr: (bsr[i], 0, 0))]
    for l in range(1, 4):
        in_specs += [wspec(l), bspec(l)]

    grid_spec = pltpu.PrefetchScalarGridSpec(
        num_scalar_prefetch=2,
        grid=(G,),
        in_specs=in_specs,
        out_specs=pl.BlockSpec((1, 1), lambda i, bsr, bvr: (0, 0)),
    )
    args = [x_sorted]
    for l in range(4):
        args += [Ws[l], Bs[l]]
    out = pl.pallas_call(
        _mlp_body, grid_spec=grid_spec,
        out_shape=jax.ShapeDtypeStruct((1, 1), jnp.float32),
    )(bs, bv, *args)
    return out.reshape(1)


# speedup vs baseline: 1.0048x; 1.0048x over previous
import jax
import jax.numpy as jnp
from jax import lax
from jax.experimental import pallas as pl
from jax.experimental.pallas import tpu as pltpu
from jax.experimental.pallas import tpu_sc as plsc

NUM_MODELS = 8
NUM_SPECIES = 4
DIMS = [384, 160, 128, 96, 1]
N = 20000
B = 1024
G = 23
TOTAL = G * B

NC, NS = 2, 16
NW = NC * NS
APW = N // NW
CH = 125
NCH = APW // CH


def _dot(a, b, prec):
    return lax.dot_general(a, b, (((1,), (0,)), ((), ())),
                           precision=prec,
                           preferred_element_type=jnp.float32)


_PREC = lax.Precision.DEFAULT


def _celu(x):
    return jnp.where(x > 0, x, jnp.exp(x) - 1.0)


def _sc_scatter_body(lin_hbm, pos_hbm, aev_hbm, out_hbm, lin_v, pos_v,
                     buf0, buf1, isem0, isem1, osem0, osem1):
    wid = lax.axis_index("s") * NC + lax.axis_index("c")
    pltpu.sync_copy(lin_hbm.at[wid], lin_v)
    pltpu.sync_copy(pos_hbm.at[wid], pos_v)
    bufs = (buf0, buf1)
    isems = (isem0, isem1)
    osems = (osem0, osem1)
    icp = [None, None]
    ocp = [None, None]
    icp[0] = pltpu.async_copy(aev_hbm.at[lin_v.at[0]], bufs[0], isems[0])
    for c in range(NCH):
        b = c & 1
        icp[b].wait()
        if c + 1 < NCH:
            if c >= 1:
                ocp[1 - b].wait()
            icp[1 - b] = pltpu.async_copy(
                aev_hbm.at[lin_v.at[c + 1]], bufs[1 - b], isems[1 - b])
        ocp[b] = pltpu.async_copy(bufs[b], out_hbm.at[pos_v.at[c]], osems[b])
    ocp[(NCH - 1) & 1].wait()
    if NCH > 1:
        ocp[NCH & 1].wait()


def _sc_scatter(lin3, pos3, aev_flat):
    fn = pl.kernel(
        _sc_scatter_body,
        mesh=plsc.VectorSubcoreMesh(core_axis_name="c", subcore_axis_name="s"),
        out_type=jax.ShapeDtypeStruct((TOTAL, DIMS[0]), jnp.float32),
        scratch_types=[
            pltpu.VMEM((NCH, CH), jnp.int32),
            pltpu.VMEM((NCH, CH), jnp.int32),
            pltpu.VMEM((CH, DIMS[0]), jnp.float32),
            pltpu.VMEM((CH, DIMS[0]), jnp.float32),
            pltpu.SemaphoreType.DMA,
            pltpu.SemaphoreType.DMA,
            pltpu.SemaphoreType.DMA,
            pltpu.SemaphoreType.DMA,
        ],
    )
    return fn(lin3, pos3, aev_flat)


def _mlp_body(bs_ref, bv_ref, x_ref, w0_ref, b0_ref, w1_ref, b1_ref,
              w2_ref, b2_ref, w3_ref, b3_ref, out_ref):
    i = pl.program_id(0)
    nv = bv_ref[i]

    @pl.when(i == 0)
    def _():
        out_ref[...] = jnp.zeros((1, 1), jnp.float32)

    @pl.when(nv > 0)
    def _():
        x = x_ref[...]
        rowmask = lax.broadcasted_iota(jnp.int32, (B, 1), 0) < nv
        h0 = _dot(x, w0_ref[0], _PREC) + b0_ref[0]
        y = jnp.zeros((B, 1), jnp.float32)
        bias_tot = jnp.float32(0.0)
        for m in range(NUM_MODELS):
            h = _celu(h0[:, m * DIMS[1]:(m + 1) * DIMS[1]])
            h = _dot(h, w1_ref[0, m], _PREC) + b1_ref[0, m]
            h = _celu(h)
            h = _dot(h, w2_ref[0, m], _PREC) + b2_ref[0, m]
            h = _celu(h)
            y = y + _dot(h, w3_ref[0, m], _PREC)
            bias_tot = bias_tot + b3_ref[0, m, 0, 0]
        ysum = jnp.sum(jnp.where(rowmask, y, 0.0), axis=(0, 1), keepdims=True)
        acc = ysum + nv.astype(jnp.float32) * bias_tot
        out_ref[...] += acc * (1.0 / NUM_MODELS)


def _routing(species):
    sf = species.reshape(-1).astype(jnp.int32)
    oh = (sf[:, None] == jnp.arange(NUM_SPECIES, dtype=jnp.int32)[None, :]).astype(jnp.int32)
    csum = jnp.cumsum(oh, axis=0)
    counts = csum[-1]
    rank = jnp.sum(oh * csum, axis=1) - 1
    nb = (counts + B - 1) // B
    blk_start = jnp.concatenate([jnp.zeros(1, jnp.int32), jnp.cumsum(nb)])[:NUM_SPECIES]
    pos = jnp.sum(oh * (blk_start * B)[None, :], axis=1) + rank
    g = jnp.arange(G, dtype=jnp.int32)
    cnb = jnp.cumsum(nb)
    bs = jnp.clip(jnp.sum((g[:, None] >= cnb[None, :]).astype(jnp.int32), axis=1),
                  0, NUM_SPECIES - 1).astype(jnp.int32)
    ohg = (bs[:, None] == jnp.arange(NUM_SPECIES, dtype=jnp.int32)[None, :]).astype(jnp.int32)
    counts_g = jnp.sum(ohg * counts[None, :], axis=1)
    start_g = jnp.sum(ohg * blk_start[None, :], axis=1)
    bv = jnp.clip(counts_g - (g - start_g) * B, 0, B).astype(jnp.int32)
    return pos, bs, bv


def kernel(species, aev,
           W0_0, b0_0, W0_1, b0_1, W0_2, b0_2, W0_3, b0_3,
           W1_0, b1_0, W1_1, b1_1, W1_2, b1_2, W1_3, b1_3,
           W2_0, b2_0, W2_1, b2_1, W2_2, b2_2, W2_3, b2_3,
           W3_0, b3_0, W3_1, b3_1, W3_2, b3_2, W3_3, b3_3):
    kw = dict(locals())
    aev_flat = aev.reshape(-1, DIMS[0])
    pos, bs, bv = _routing(species)

    lin = jnp.arange(N, dtype=jnp.int32).reshape(NW, NCH, CH)
    x_sorted = _sc_scatter(lin, pos.reshape(NW, NCH, CH), aev_flat)

    bs, bv, *wflat = lax.optimization_barrier(
        (bs, bv) + tuple(kw["W%d_%d" % (s, l)]
                         for l in range(4) for s in range(NUM_SPECIES))
        + tuple(kw["b%d_%d" % (s, l)]
                for l in range(4) for s in range(NUM_SPECIES)))
    for idx_l in range(4):
        for idx_s in range(NUM_SPECIES):
            kw["W%d_%d" % (idx_s, idx_l)] = wflat[idx_l * NUM_SPECIES + idx_s]
            kw["b%d_%d" % (idx_s, idx_l)] = wflat[16 + idx_l * NUM_SPECIES + idx_s]

    Ws = [jnp.stack([kw["W%d_%d" % (s, l)] for s in range(NUM_SPECIES)])
          for l in range(4)]
    Bs = [jnp.stack([kw["b%d_%d" % (s, l)] for s in range(NUM_SPECIES)])
          for l in range(4)]
    Ws[0] = Ws[0].transpose(0, 2, 1, 3).reshape(NUM_SPECIES, DIMS[0],
                                                NUM_MODELS * DIMS[1])
    Bs[0] = Bs[0].reshape(NUM_SPECIES, 1, NUM_MODELS * DIMS[1])

    def wspec(l):
        return pl.BlockSpec((1, NUM_MODELS, DIMS[l], DIMS[l + 1]),
                            lambda i, bsr, bvr: (bsr[i], 0, 0, 0))

    def bspec(l):
        return pl.BlockSpec((1, NUM_MODELS, 1, DIMS[l + 1]),
                            lambda i, bsr, bvr: (bsr[i], 0, 0, 0))

    in_specs = [pl.BlockSpec((B, DIMS[0]), lambda i, bsr, bvr: (i, 0)),
                pl.BlockSpec((1, DIMS[0], NUM_MODELS * DIMS[1]),
                             lambda i, bsr, bvr: (bsr[i], 0, 0)),
                pl.BlockSpec((1, 1, NUM_MODELS * DIMS[1]),
                             lambda i, bsr, bvr: (bsr[i], 0, 0))]
    for l in range(1, 4):
        in_specs += [wspec(l), bspec(l)]

    grid_spec = pltpu.PrefetchScalarGridSpec(
        num_scalar_prefetch=2,
        grid=(G,),
        in_specs=in_specs,
        out_specs=pl.BlockSpec((1, 1), lambda i, bsr, bvr: (0, 0)),
    )
    args = [x_sorted]
    for l in range(4):
        args += [Ws[l], Bs[l]]
    out = pl.pallas_call(
        _mlp_body, grid_spec=grid_spec,
        out_shape=jax.ShapeDtypeStruct((1, 1), jnp.float32),
    )(bs, bv, *args)
    return out.reshape(1)
